# baseline (speedup 1.0000x reference)
import jax
import jax.numpy as jnp
from jax import lax
from jax.experimental import pallas as pl
from jax.experimental.pallas import tpu as pltpu
from jax.experimental.pallas import tpu_sc as plsc

N = 10000
NPAD = 10240
F = 256
H = 128
NS = 16
NC = 2
EPT = 10240
EB = 128
NB = EPT // EB
NBUF = 4
ROWS_PT = NPAD // NS
TRASH = N
PK = 16384


def _mesh():
    return plsc.VectorSubcoreMesh(core_axis_name="c", subcore_axis_name="s")


_SC_PARAMS = pltpu.CompilerParams(needs_layout_passes=False,
                                  use_tc_tiling_on_sc=False)


def _deg_body(pk_hbm, deg_out, pk_v, hist, tmp, red, hist_sh):
    c = lax.axis_index("c")
    s = lax.axis_index("s")

    pltpu.sync_copy(pk_hbm.at[pl.ds(s * NB + c * (NB // 2), NB // 2)], pk_v)

    def zero_hist(i, _):
        hist[pl.ds(i * 16, 16)] = jnp.zeros((16,), jnp.float32)
        return ()
    lax.fori_loop(0, NPAD // 16, zero_hist, ())

    ones = jnp.ones((16,), jnp.float32)

    def batch_body(j, _):
        for v in range(EB // 16):
            idx = lax.bitwise_and(pk_v[j, pl.ds(v * 16, 16)], PK - 1)
            plsc.addupdate_scatter(hist, [idx], ones)
        return ()
    lax.fori_loop(0, NB // 2, batch_body, ())

    pltpu.sync_copy(hist, hist_sh.at[pl.ds(s * NPAD, NPAD)])
    plsc.subcore_barrier()

    base = s * ROWS_PT

    def zero_red(i, _):
        red[pl.ds(i * 16, 16)] = jnp.zeros((16,), jnp.float32)
        return ()
    lax.fori_loop(0, ROWS_PT // 16, zero_red, ())

    for t in range(NS):
        pltpu.sync_copy(hist_sh.at[pl.ds(t * NPAD + base, ROWS_PT)], tmp)

        def acc_body(i, _):
            sl = pl.ds(i * 16, 16)
            red[sl] = red[sl] + tmp[sl]
            return ()
        lax.fori_loop(0, ROWS_PT // 16, acc_body, ())

    pltpu.sync_copy(red, deg_out.at[pl.ds(c * NPAD + base, ROWS_PT)])


def _deg_call(pk_flat):
    return pl.kernel(
        _deg_body,
        out_type=jax.ShapeDtypeStruct((NC * NPAD,), jnp.float32),
        mesh=_mesh(),
        scratch_types=[
            pltpu.VMEM((NB // 2, EB), jnp.int32),
            pltpu.VMEM((NPAD,), jnp.float32),
            pltpu.VMEM((ROWS_PT,), jnp.float32),
            pltpu.VMEM((ROWS_PT,), jnp.float32),
            pltpu.VMEM_SHARED((NS * NPAD,), jnp.float32),
        ],
        compiler_params=_SC_PARAMS,
    )(pk_flat)


def _agg_body(xs_hbm, pk_hbm, out_hbm,
              pk_v, sidx, didx, gbuf, acc,
              gsem0, gsem1, gsem2, gsem3, ssem0, ssem1, ssem2, ssem3):
    c = lax.axis_index("c")
    s = lax.axis_index("s")
    gsems = (gsem0, gsem1, gsem2, gsem3)
    ssems = (ssem0, ssem1, ssem2, ssem3)

    pltpu.sync_copy(pk_hbm.at[pl.ds(s * NB, NB)], pk_v)

    base = s * ROWS_PT
    pltpu.sync_copy(xs_hbm.at[pl.ds(c * NPAD + base, ROWS_PT)],
                    acc.at[pl.ds(base, ROWS_PT)])
    plsc.subcore_barrier()

    cbase = c * NPAD

    def unpack(bi, b):
        for v in range(EB // 16):
            sl = pl.ds(v * 16, 16)
            p = pk_v[bi, sl]
            sidx[b, sl] = lax.shift_right_logical(p, 14) + cbase
            didx[b, sl] = lax.bitwise_and(p, PK - 1)

    def start_gather(bi, b):
        unpack(bi, b)
        pltpu.async_copy(xs_hbm.at[sidx.at[b]], gbuf.at[b], gsems[b])

    def wait_gather(b):
        pltpu.make_async_copy(xs_hbm.at[sidx.at[b]], gbuf.at[b],
                              gsems[b]).wait()

    def start_scatter(b):
        pltpu.async_copy(gbuf.at[b], acc.at[didx.at[b]], ssems[b], add=True)

    def wait_scatter(b):
        pltpu.make_async_copy(gbuf.at[b], acc.at[didx.at[b]],
                              ssems[b]).wait()

    for b in range(2):
        start_gather(b, b)

    def loop_body(k, _):
        for j in range(NBUF):
            bi = k * NBUF + j
            wait_gather(j)
            start_scatter(j)
            jf = (j + 2) % NBUF
            nxt = bi + 2

            @pl.when(nxt < NB)
            def _():
                @pl.when(bi >= 2)
                def _():
                    wait_scatter(jf)
                start_gather(nxt, jf)
        return ()
    lax.fori_loop(0, NB // NBUF, loop_body, ())

    for b in range(NBUF):
        wait_scatter(b)

    plsc.subcore_barrier()
    pltpu.sync_copy(acc.at[pl.ds(base, ROWS_PT)],
                    out_hbm.at[pl.ds(c * NPAD + base, ROWS_PT)])


def _agg_call(xs_flat, pk_flat):
    return pl.kernel(
        _agg_body,
        out_type=jax.ShapeDtypeStruct((NC * NPAD, H), jnp.bfloat16),
        mesh=_mesh(),
        scratch_types=[
            pltpu.VMEM((NB, EB), jnp.int32),
            pltpu.VMEM((NBUF, EB), jnp.int32),
            pltpu.VMEM((NBUF, EB), jnp.int32),
            pltpu.VMEM((NBUF, EB, H), jnp.bfloat16),
            pltpu.VMEM_SHARED((NPAD, H), jnp.bfloat16),
        ] + [pltpu.SemaphoreType.DMA] * (2 * NBUF),
        compiler_params=_SC_PARAMS,
    )(xs_flat, pk_flat)


_RB = 1024


def _prep_body(deg_ref, x_ref, xs_ref, dinv_ref):
    deg = deg_ref[0] + deg_ref[1] + 1.0
    dinv = lax.rsqrt(deg)
    dinv_ref[...] = dinv
    xb = x_ref[...]
    xs_ref[0] = (xb[:, :H] * dinv).astype(jnp.bfloat16)
    xs_ref[1] = (xb[:, H:] * dinv).astype(jnp.bfloat16)


def _prep_call(deg2, x_pad):
    return pl.pallas_call(
        _prep_body,
        grid=(NPAD // _RB,),
        in_specs=[
            pl.BlockSpec((2, _RB, 1), lambda i: (0, i, 0)),
            pl.BlockSpec((_RB, F), lambda i: (i, 0)),
        ],
        out_specs=[
            pl.BlockSpec((2, _RB, H), lambda i: (0, i, 0)),
            pl.BlockSpec((_RB, 1), lambda i: (i, 0)),
        ],
        out_shape=[
            jax.ShapeDtypeStruct((2, NPAD, H), jnp.bfloat16),
            jax.ShapeDtypeStruct((NPAD, 1), jnp.float32),
        ],
    )(deg2, x_pad)


def _mid_body(a_ref, dinv_ref, w1_ref, b1_ref, w2_ref, o_ref):
    dv = dinv_ref[...]
    a0 = a_ref[0].astype(jnp.float32) * dv
    a1 = a_ref[1].astype(jnp.float32) * dv
    h1 = jnp.dot(a0, w1_ref[0], preferred_element_type=jnp.float32)
    h1 = h1 + jnp.dot(a1, w1_ref[1], preferred_element_type=jnp.float32)
    h1 = jnp.maximum(h1 + b1_ref[...], 0.0)
    t = jnp.dot(h1, w2_ref[...], preferred_element_type=jnp.float32)
    o_ref[0] = (t[:, :H] * dv).astype(jnp.bfloat16)
    o_ref[1] = (t[:, H:] * dv).astype(jnp.bfloat16)


def _mid_call(agg1, dinv, w1r, b1r, w2):
    return pl.pallas_call(
        _mid_body,
        grid=(NPAD // _RB,),
        in_specs=[
            pl.BlockSpec((2, _RB, H), lambda i: (0, i, 0)),
            pl.BlockSpec((_RB, 1), lambda i: (i, 0)),
            pl.BlockSpec((2, H, 512), lambda i: (0, 0, 0)),
            pl.BlockSpec((1, 512), lambda i: (0, 0)),
            pl.BlockSpec((512, F), lambda i: (0, 0)),
        ],
        out_specs=pl.BlockSpec((2, _RB, H), lambda i: (0, i, 0)),
        out_shape=jax.ShapeDtypeStruct((2, NPAD, H), jnp.bfloat16),
    )(agg1, dinv, w1r, b1r, w2)


def _fin_body(a_ref, dinv_ref, b2_ref, o_ref):
    dv = dinv_ref[...]
    a0 = a_ref[0].astype(jnp.float32) * dv
    a1 = a_ref[1].astype(jnp.float32) * dv
    o_ref[...] = jnp.concatenate([a0, a1], axis=1) + b2_ref[...]


def _fin_call(agg2, dinv, b2r):
    return pl.pallas_call(
        _fin_body,
        grid=(NPAD // _RB,),
        in_specs=[
            pl.BlockSpec((2, _RB, H), lambda i: (0, i, 0)),
            pl.BlockSpec((_RB, 1), lambda i: (i, 0)),
            pl.BlockSpec((1, F), lambda i: (0, 0)),
        ],
        out_specs=pl.BlockSpec((_RB, F), lambda i: (i, 0)),
        out_shape=jax.ShapeDtypeStruct((NPAD, F), jnp.float32),
    )(agg2, dinv, b2r)


def kernel(x, edge_index, W1, b1, W2, b2):
    E = edge_index.shape[1]
    src = edge_index[0].astype(jnp.int32)
    dst = edge_index[1].astype(jnp.int32)

    tot = NS * EPT
    pad = tot - E
    pk = src * PK + dst
    pk_p = jnp.concatenate([pk, jnp.full((pad,), TRASH, jnp.int32)])
    pk_flat = pk_p.reshape(NS * NB, EB)

    x_pad = jnp.pad(x, ((0, NPAD - N), (0, 0)))
    w1r = W1.reshape(2, H, 512)
    b1r = b1.reshape(1, 512)
    b2r = b2.reshape(1, F)

    deg = _deg_call(pk_flat).reshape(NC, NPAD, 1)
    xs1, dinv = _prep_call(deg, x_pad)
    agg1 = _agg_call(xs1.reshape(NC * NPAD, H), pk_flat)
    xs2 = _mid_call(agg1.reshape(NC, NPAD, H), dinv, w1r, b1r, W2)
    agg2 = _agg_call(xs2.reshape(NC * NPAD, H), pk_flat)
    out = _fin_call(agg2.reshape(NC, NPAD, H), dinv, b2r)
    return out[:N]


# speedup vs baseline: 1.6485x; 1.6485x over previous
import jax
import jax.numpy as jnp
from jax import lax
from jax.experimental import pallas as pl
from jax.experimental.pallas import tpu as pltpu
from jax.experimental.pallas import tpu_sc as plsc

N = 10000
NPAD = 10240
F = 256
H = 128
NS = 16
NC = 2
EPT = 10240
EB = 128
NB = EPT // EB
NBUF = 4
ROWS_PT = NPAD // NS
TRASH = N
PK = 16384


def _mesh():
    return plsc.VectorSubcoreMesh(core_axis_name="c", subcore_axis_name="s")


_SC_PARAMS = pltpu.CompilerParams(needs_layout_passes=False,
                                  use_tc_tiling_on_sc=False)


def _deg_body(pk_hbm, deg_out, pk_v, hist, tmp, red, hist_sh):
    c = lax.axis_index("c")
    s = lax.axis_index("s")

    pltpu.sync_copy(pk_hbm.at[pl.ds(s * NB + c * (NB // 2), NB // 2)], pk_v)

    def zero_hist(i, _):
        hist[pl.ds(i * 16, 16)] = jnp.zeros((16,), jnp.float32)
        return ()
    lax.fori_loop(0, NPAD // 16, zero_hist, ())

    ones = jnp.ones((16,), jnp.float32)

    def batch_body(j, _):
        for v in range(EB // 16):
            idx = lax.bitwise_and(pk_v[j, pl.ds(v * 16, 16)], PK - 1)
            plsc.addupdate_scatter(hist, [idx], ones)
        return ()
    lax.fori_loop(0, NB // 2, batch_body, ())

    pltpu.sync_copy(hist, hist_sh.at[pl.ds(s * NPAD, NPAD)])
    plsc.subcore_barrier()

    base = s * ROWS_PT

    def zero_red(i, _):
        red[pl.ds(i * 16, 16)] = jnp.zeros((16,), jnp.float32)
        return ()
    lax.fori_loop(0, ROWS_PT // 16, zero_red, ())

    for t in range(NS):
        pltpu.sync_copy(hist_sh.at[pl.ds(t * NPAD + base, ROWS_PT)], tmp)

        def acc_body(i, _):
            sl = pl.ds(i * 16, 16)
            red[sl] = red[sl] + tmp[sl]
            return ()
        lax.fori_loop(0, ROWS_PT // 16, acc_body, ())

    pltpu.sync_copy(red, deg_out.at[pl.ds(c * NPAD + base, ROWS_PT)])


def _deg_call(pk_flat):
    return pl.kernel(
        _deg_body,
        out_type=jax.ShapeDtypeStruct((NC * NPAD,), jnp.float32),
        mesh=_mesh(),
        scratch_types=[
            pltpu.VMEM((NB // 2, EB), jnp.int32),
            pltpu.VMEM((NPAD,), jnp.float32),
            pltpu.VMEM((ROWS_PT,), jnp.float32),
            pltpu.VMEM((ROWS_PT,), jnp.float32),
            pltpu.VMEM_SHARED((NS * NPAD,), jnp.float32),
        ],
        compiler_params=_SC_PARAMS,
    )(pk_flat)


def _agg_body(xs0_hbm, xs1_hbm, pk_hbm, out0_hbm, out1_hbm,
              pk_v, sidx, didx, gbuf, xs_sh, acc,
              gsem0, gsem1, gsem2, gsem3, ssem0, ssem1, ssem2, ssem3):
    c = lax.axis_index("c")
    s = lax.axis_index("s")
    gsems = (gsem0, gsem1, gsem2, gsem3)
    ssems = (ssem0, ssem1, ssem2, ssem3)

    pltpu.sync_copy(pk_hbm.at[pl.ds(s * NB, NB)], pk_v)

    base = s * ROWS_PT
    rows = pl.ds(base, ROWS_PT)

    @pl.when(c == 0)
    def _():
        pltpu.sync_copy(xs0_hbm.at[rows], xs_sh.at[rows])
        pltpu.sync_copy(xs0_hbm.at[rows], acc.at[rows])

    @pl.when(c == 1)
    def _():
        pltpu.sync_copy(xs1_hbm.at[rows], xs_sh.at[rows])
        pltpu.sync_copy(xs1_hbm.at[rows], acc.at[rows])

    plsc.subcore_barrier()

    def unpack(bi, b):
        for v in range(EB // 16):
            sl = pl.ds(v * 16, 16)
            p = pk_v[bi, sl]
            sidx[b, sl] = lax.shift_right_logical(p, 14)
            didx[b, sl] = lax.bitwise_and(p, PK - 1)

    def start_gather(bi, b):
        unpack(bi, b)
        pltpu.async_copy(xs_sh.at[sidx.at[b]], gbuf.at[b], gsems[b])

    def wait_gather(b):
        pltpu.make_async_copy(xs_sh.at[sidx.at[b]], gbuf.at[b],
                              gsems[b]).wait()

    def start_scatter(b):
        pltpu.async_copy(gbuf.at[b], acc.at[didx.at[b]], ssems[b], add=True)

    def wait_scatter(b):
        pltpu.make_async_copy(gbuf.at[b], acc.at[didx.at[b]],
                              ssems[b]).wait()

    for b in range(2):
        start_gather(b, b)

    def loop_body(k, _):
        for j in range(NBUF):
            bi = k * NBUF + j
            wait_gather(j)
            start_scatter(j)
            jf = (j + 2) % NBUF
            nxt = bi + 2

            @pl.when(nxt < NB)
            def _():
                @pl.when(bi >= 2)
                def _():
                    wait_scatter(jf)
                start_gather(nxt, jf)
        return ()
    lax.fori_loop(0, NB // NBUF, loop_body, ())

    for b in range(NBUF):
        wait_scatter(b)

    plsc.subcore_barrier()

    @pl.when(c == 0)
    def _():
        pltpu.sync_copy(acc.at[rows], out0_hbm.at[rows])

    @pl.when(c == 1)
    def _():
        pltpu.sync_copy(acc.at[rows], out1_hbm.at[rows])


def _agg_call(xs0, xs1, pk_flat):
    return pl.kernel(
        _agg_body,
        out_type=[jax.ShapeDtypeStruct((NPAD, H), jnp.bfloat16),
                  jax.ShapeDtypeStruct((NPAD, H), jnp.bfloat16)],
        mesh=_mesh(),
        scratch_types=[
            pltpu.VMEM((NB, EB), jnp.int32),
            pltpu.VMEM((NBUF, EB), jnp.int32),
            pltpu.VMEM((NBUF, EB), jnp.int32),
            pltpu.VMEM((NBUF, EB, H), jnp.bfloat16),
            pltpu.VMEM_SHARED((NPAD, H), jnp.bfloat16),
            pltpu.VMEM_SHARED((NPAD, H), jnp.bfloat16),
        ] + [pltpu.SemaphoreType.DMA] * (2 * NBUF),
        compiler_params=_SC_PARAMS,
    )(xs0, xs1, pk_flat)


_RB = 1024


def _prep_body(dega_ref, degb_ref, x_ref, xs0_ref, xs1_ref, dinv_ref):
    deg = (dega_ref[...] + degb_ref[...] + 1.0).reshape(_RB, 1)
    dinv = lax.rsqrt(deg)
    dinv_ref[...] = dinv
    xb = x_ref[...]
    xs0_ref[...] = (xb[:, :H] * dinv).astype(jnp.bfloat16)
    xs1_ref[...] = (xb[:, H:] * dinv).astype(jnp.bfloat16)


def _prep_call(deg2, x_pad):
    return pl.pallas_call(
        _prep_body,
        grid=(NPAD // _RB,),
        in_specs=[
            pl.BlockSpec((_RB,), lambda i: (i,)),
            pl.BlockSpec((_RB,), lambda i: (NPAD // _RB + i,)),
            pl.BlockSpec((_RB, F), lambda i: (i, 0)),
        ],
        out_specs=[
            pl.BlockSpec((_RB, H), lambda i: (i, 0)),
            pl.BlockSpec((_RB, H), lambda i: (i, 0)),
            pl.BlockSpec((_RB, 1), lambda i: (i, 0)),
        ],
        out_shape=[
            jax.ShapeDtypeStruct((NPAD, H), jnp.bfloat16),
            jax.ShapeDtypeStruct((NPAD, H), jnp.bfloat16),
            jax.ShapeDtypeStruct((NPAD, 1), jnp.float32),
        ],
    )(deg2, deg2, x_pad)


def _mid_body(a0_ref, a1_ref, dinv_ref, w1_ref, b1_ref, w2_ref,
              o0_ref, o1_ref):
    dv = dinv_ref[...]
    a0 = a0_ref[...].astype(jnp.float32) * dv
    a1 = a1_ref[...].astype(jnp.float32) * dv
    h1 = jnp.dot(a0, w1_ref[0], preferred_element_type=jnp.float32)
    h1 = h1 + jnp.dot(a1, w1_ref[1], preferred_element_type=jnp.float32)
    h1 = jnp.maximum(h1 + b1_ref[...], 0.0)
    t = jnp.dot(h1, w2_ref[...], preferred_element_type=jnp.float32)
    o0_ref[...] = (t[:, :H] * dv).astype(jnp.bfloat16)
    o1_ref[...] = (t[:, H:] * dv).astype(jnp.bfloat16)


def _mid_call(a0, a1, dinv, w1r, b1r, w2):
    return pl.pallas_call(
        _mid_body,
        grid=(NPAD // _RB,),
        in_specs=[
            pl.BlockSpec((_RB, H), lambda i: (i, 0)),
            pl.BlockSpec((_RB, H), lambda i: (i, 0)),
            pl.BlockSpec((_RB, 1), lambda i: (i, 0)),
            pl.BlockSpec((2, H, 512), lambda i: (0, 0, 0)),
            pl.BlockSpec((1, 512), lambda i: (0, 0)),
            pl.BlockSpec((512, F), lambda i: (0, 0)),
        ],
        out_specs=[
            pl.BlockSpec((_RB, H), lambda i: (i, 0)),
            pl.BlockSpec((_RB, H), lambda i: (i, 0)),
        ],
        out_shape=[
            jax.ShapeDtypeStruct((NPAD, H), jnp.bfloat16),
            jax.ShapeDtypeStruct((NPAD, H), jnp.bfloat16),
        ],
    )(a0, a1, dinv, w1r, b1r, w2)


def _fin_body(a0_ref, a1_ref, dinv_ref, b2_ref, o_ref):
    dv = dinv_ref[...]
    a0 = a0_ref[...].astype(jnp.float32) * dv
    a1 = a1_ref[...].astype(jnp.float32) * dv
    o_ref[...] = jnp.concatenate([a0, a1], axis=1) + b2_ref[...]


def _fin_call(a0, a1, dinv, b2r):
    return pl.pallas_call(
        _fin_body,
        grid=(NPAD // _RB,),
        in_specs=[
            pl.BlockSpec((_RB, H), lambda i: (i, 0)),
            pl.BlockSpec((_RB, H), lambda i: (i, 0)),
            pl.BlockSpec((_RB, 1), lambda i: (i, 0)),
            pl.BlockSpec((1, F), lambda i: (0, 0)),
        ],
        out_specs=pl.BlockSpec((_RB, F), lambda i: (i, 0)),
        out_shape=jax.ShapeDtypeStruct((NPAD, F), jnp.float32),
    )(a0, a1, dinv, b2r)


def kernel(x, edge_index, W1, b1, W2, b2):
    E = edge_index.shape[1]
    src = edge_index[0].astype(jnp.int32)
    dst = edge_index[1].astype(jnp.int32)

    tot = NS * EPT
    pad = tot - E
    pk = src * PK + dst
    pk_p = jnp.concatenate([pk, jnp.full((pad,), TRASH, jnp.int32)])
    pk_flat = pk_p.reshape(NS * NB, EB)

    x_pad = jnp.pad(x, ((0, NPAD - N), (0, 0)))
    w1r = W1.reshape(2, H, 512)
    b1r = b1.reshape(1, 512)
    b2r = b2.reshape(1, F)

    deg = _deg_call(pk_flat)
    xs0, xs1, dinv = _prep_call(deg, x_pad)
    a0, a1 = _agg_call(xs0, xs1, pk_flat)
    m0, m1 = _mid_call(a0, a1, dinv, w1r, b1r, W2)
    g0, g1 = _agg_call(m0, m1, pk_flat)
    out = _fin_call(g0, g1, dinv, b2r)
    return out[:N]
